# baseline (speedup 1.0000x reference)
import functools

import jax
import jax.numpy as jnp
from jax import lax
from jax.experimental import pallas as pl
from jax.experimental.pallas import tpu as pltpu
from jax.experimental.pallas import tpu_sc as plsc

N_NODES = 10000
N_EDGES = 160000
CH = 128
E_BLK = 80
NS = 16
ACC_ROWS = 10240
CHUNKS_A = 45
CHUNKS_B = 80
E_A = CHUNKS_A * E_BLK * NS
E_B = CHUNKS_B * E_BLK * NS


def _gates_body(attr_ref, wT_ref, b_ref, out_ref):
    z = jnp.dot(attr_ref[...], wT_ref[0], preferred_element_type=jnp.float32)
    gp = jax.nn.sigmoid(z + b_ref[0])
    ib = jax.lax.bitcast_convert_type(gp, jnp.int32) + 0x8000
    lo = ib[:, :64] >> 16
    hi = ib[:, 64:] & (-65536)
    out_ref[...] = lo | hi


def _gates_tc(edge_attr, WgT_s, bg_s, bm):
    ne = edge_attr.shape[0]
    nm = ne // bm
    return pl.pallas_call(
        _gates_body,
        grid=(2, nm),
        in_specs=[
            pl.BlockSpec((bm, 16), lambda h, m: (m, 0)),
            pl.BlockSpec((1, 16, CH), lambda h, m: (h, 0, 0)),
            pl.BlockSpec((1, 1, CH), lambda h, m: (h, 0, 0)),
        ],
        out_specs=pl.BlockSpec((bm, CH // 2), lambda h, m: (h * nm + m, 0)),
        out_shape=jax.ShapeDtypeStruct((2 * ne, CH // 2), jnp.int32),
        compiler_params=pltpu.CompilerParams(
            dimension_semantics=("parallel", "arbitrary")),
    )(edge_attr, WgT_s, bg_s)


def _make_sc_body(chunks):
    n_edges_h = chunks * E_BLK * NS
    ept = chunks * E_BLK

    def body(comb_hbm, x2_hbm, gates_hbm, out_hbm,
             idx_v, rows0, rows1, g0, g1, acc,
             lsem0, lsem1, ssem0, ssem1, isem0, isem1, isem2, isem3):
        c = lax.axis_index("c")
        s = lax.axis_index("s")
        rows = (rows0, rows1)
        g = (g0, g1)
        lsem = (lsem0, lsem1)
        ssem = (ssem0, ssem1)
        isem = (isem0, isem1, isem2, isem3)

        def idx_load(k, slot):
            pltpu.async_copy(comb_hbm.at[c, s * chunks + k],
                             idx_v.at[pl.ds(2 * slot, 2)], isem[slot])

        def idx_wait(k, slot):
            pltpu.make_async_copy(comb_hbm.at[c, s * chunks + k],
                                  idx_v.at[pl.ds(2 * slot, 2)],
                                  isem[slot]).wait()

        def gbase(k):
            return c * n_edges_h + s * ept + k * E_BLK

        def load_start(k, slot, b):
            pltpu.async_copy(x2_hbm.at[idx_v.at[2 * slot]], rows[b], lsem[b])
            pltpu.async_copy(gates_hbm.at[pl.ds(gbase(k), E_BLK)], g[b],
                             lsem[b])

        def load_wait(k, slot, b):
            pltpu.make_async_copy(x2_hbm.at[idx_v.at[2 * slot]], rows[b],
                                  lsem[b]).wait()
            pltpu.make_async_copy(gates_hbm.at[pl.ds(gbase(k), E_BLK)], g[b],
                                  lsem[b]).wait()

        def scat_start(slot, b):
            pltpu.async_copy(rows[b], acc.at[idx_v.at[2 * slot + 1]], ssem[b],
                             add=True)

        def scat_wait(slot, b):
            pltpu.make_async_copy(rows[b], acc.at[idx_v.at[2 * slot + 1]],
                                  ssem[b]).wait()

        def multiply(b):
            rv, gv = rows[b], g[b]

            @plsc.parallel_loop(0, E_BLK, unroll=2)
            def _(e):
                for t in range(CH // 32):
                    w = gv[e, pl.ds(t * 16, 16)]
                    ga = jax.lax.bitcast_convert_type(w << 16, jnp.float32)
                    gb = jax.lax.bitcast_convert_type(w & -65536, jnp.float32)
                    sl0 = pl.ds(t * 32, 16)
                    sl1 = pl.ds(t * 32 + 16, 16)
                    rv[e, sl0] = rv[e, sl0] * ga
                    rv[e, sl1] = rv[e, sl1] * gb

        @plsc.parallel_loop(0, E_BLK, unroll=2)
        def _(e):
            for j in range(CH // 16):
                rows0[e, pl.ds(j * 16, 16)] = jnp.zeros((16,), jnp.float32)

        def zacc(k, _):
            pltpu.sync_copy(rows0, acc.at[pl.ds(s * 640 + k * E_BLK, E_BLK)])
            return 0
        lax.fori_loop(0, ACC_ROWS // NS // E_BLK, zacc, 0)
        plsc.subcore_barrier()

        pltpu.sync_copy(comb_hbm.at[c, s * chunks], idx_v.at[pl.ds(0, 2)])
        idx_load(1, 1)
        load_start(0, 0, 0)

        def substep(k, b4):
            b = b4 % 2
            load_wait(k, b4, b)

            @pl.when(k > 0)
            def _():
                scat_wait((b4 + 3) % 4, 1 - b)

            @pl.when(k < chunks - 2)
            def _():
                idx_load(k + 2, (b4 + 2) % 4)

            idx_wait(k + 1, (b4 + 1) % 4)
            load_start(k + 1, (b4 + 1) % 4, 1 - b)
            multiply(b)
            scat_start(b4, b)

        nq = (chunks - 1) // 4
        rem = (chunks - 1) - 4 * nq

        def quad(q, _):
            for b4 in range(4):
                substep(4 * q + b4, b4)
            return 0
        lax.fori_loop(0, nq, quad, 0)
        for b4 in range(rem):
            substep(4 * nq + b4, b4)

        kt = chunks - 1
        load_wait(kt, kt % 4, kt % 2)
        scat_wait((kt + 3) % 4, 1 - kt % 2)
        multiply(kt % 2)
        scat_start(kt % 4, kt % 2)
        scat_wait(kt % 4, kt % 2)
        plsc.subcore_barrier()

        @pl.when(s < NS - 1)
        def _():
            off = s * 624
            pltpu.sync_copy(acc.at[pl.ds(off, 624)],
                            out_hbm.at[pl.ds(c * N_NODES + off, 624)])

        @pl.when(s == NS - 1)
        def _():
            pltpu.sync_copy(acc.at[pl.ds(9360, 640)],
                            out_hbm.at[pl.ds(c * N_NODES + 9360, 640)])

    return body


def _sc_agg(comb, x2, gates, chunks):
    mesh = plsc.VectorSubcoreMesh(core_axis_name="c", subcore_axis_name="s")
    fn = functools.partial(
        pl.kernel,
        mesh=mesh,
        out_type=jax.ShapeDtypeStruct((2 * N_NODES, CH), jnp.float32),
        scratch_types=[
            pltpu.VMEM((8, E_BLK), jnp.int32),
            pltpu.VMEM((E_BLK, CH), jnp.float32),
            pltpu.VMEM((E_BLK, CH), jnp.float32),
            pltpu.VMEM((E_BLK, CH // 2), jnp.int32),
            pltpu.VMEM((E_BLK, CH // 2), jnp.int32),
            pltpu.VMEM_SHARED((ACC_ROWS, CH), jnp.float32),
        ] + [pltpu.SemaphoreType.DMA] * 8,
    )(_make_sc_body(chunks))
    return fn(comb, x2, gates)


def _combine_body(x_ref, a0a_ref, a1a_ref, a0b_ref, a1b_ref,
                  wc_ref, b_ref, out_ref):
    dn = (((1,), (1,)), ((), ()))
    w = wc_ref[...]
    acc = jax.lax.dot_general(x_ref[...], w[:, :256], dn,
                              preferred_element_type=jnp.float32)
    acc += jax.lax.dot_general(a0a_ref[...] + a0b_ref[...],
                               w[:, 256:256 + CH], dn,
                               preferred_element_type=jnp.float32)
    acc += jax.lax.dot_general(a1a_ref[...] + a1b_ref[...],
                               w[:, 256 + CH:], dn,
                               preferred_element_type=jnp.float32)
    out_ref[...] = acc + b_ref[...]


def _combine_tc(x, agg_a, agg_b, Wc, bc2):
    BM = 1000
    nb = N_NODES // BM
    return pl.pallas_call(
        _combine_body,
        grid=(nb,),
        in_specs=[
            pl.BlockSpec((BM, 256), lambda m: (m, 0)),
            pl.BlockSpec((BM, CH), lambda m: (m, 0)),
            pl.BlockSpec((BM, CH), lambda m: (m + nb, 0)),
            pl.BlockSpec((BM, CH), lambda m: (m, 0)),
            pl.BlockSpec((BM, CH), lambda m: (m + nb, 0)),
            pl.BlockSpec((256, 512), lambda m: (0, 0)),
            pl.BlockSpec((1, 256), lambda m: (0, 0)),
        ],
        out_specs=pl.BlockSpec((BM, 256), lambda m: (m, 0)),
        out_shape=jax.ShapeDtypeStruct((N_NODES, 256), jnp.float32),
    )(x, agg_a, agg_a, agg_b, agg_b, Wc, bc2)


def _edge_prep(src_h, dst_h):
    n = src_h.shape[0] // E_BLK
    srcg = (2 * src_h[None, :] + jnp.arange(2)[:, None]).reshape(2, n, 1, E_BLK)
    dstg = jnp.broadcast_to(dst_h, (2, src_h.shape[0])).reshape(2, n, 1, E_BLK)
    return jnp.concatenate([srcg, dstg], axis=2)


def kernel(x, edge_index, edge_attr, Wg, bg, Wc, bc):
    src = edge_index[0].astype(jnp.int32)
    dst = edge_index[1].astype(jnp.int32)
    x2 = x.reshape(2 * N_NODES, CH)

    wi = jnp.arange(CH // 2)
    base = 32 * (wi // 16) + wi % 16
    perm = jnp.concatenate([base, base + 16])
    WgT_s = jnp.stack([Wg[:CH].T, Wg[CH:].T])[:, :, perm]
    bg_s = jnp.stack([bg[:CH], bg[CH:]])[:, perm].reshape(2, 1, CH)

    comb_a = _edge_prep(src[:E_A], dst[:E_A])
    comb_b = _edge_prep(src[E_A:], dst[E_A:])

    gates_a = _gates_tc(edge_attr[:E_A], WgT_s, bg_s, 3600)
    gates_b = _gates_tc(edge_attr[E_A:], WgT_s, bg_s, 6400)

    agg_a = _sc_agg(comb_a, x2, gates_a, CHUNKS_A)
    agg_b = _sc_agg(comb_b, x2, gates_b, CHUNKS_B)

    return _combine_tc(x, agg_a, agg_b, Wc, bc.reshape(1, 256))


# speedup vs baseline: 1.0419x; 1.0419x over previous
import functools

import jax
import jax.numpy as jnp
from jax import lax
from jax.experimental import pallas as pl
from jax.experimental.pallas import tpu as pltpu
from jax.experimental.pallas import tpu_sc as plsc

N_NODES = 10000
N_EDGES = 160000
CH = 128
E_BLK = 80
NS = 16
ACC_ROWS = 10240
CHUNKS_A = 66
CHUNKS_B = 59
E_A = CHUNKS_A * E_BLK * NS
E_B = CHUNKS_B * E_BLK * NS


def _gates_body(attr_ref, wT_ref, b_ref, out_ref):
    z = jnp.dot(attr_ref[...], wT_ref[0], preferred_element_type=jnp.float32)
    gp = jax.nn.sigmoid(z + b_ref[0])
    ib = jax.lax.bitcast_convert_type(gp, jnp.int32) + 0x8000
    lo = ib[:, :64] >> 16
    hi = ib[:, 64:] & (-65536)
    out_ref[...] = lo | hi


def _gates_tc(edge_attr, WgT_s, bg_s, bm):
    ne = edge_attr.shape[0]
    nm = ne // bm
    return pl.pallas_call(
        _gates_body,
        grid=(2, nm),
        in_specs=[
            pl.BlockSpec((bm, 16), lambda h, m: (m, 0)),
            pl.BlockSpec((1, 16, CH), lambda h, m: (h, 0, 0)),
            pl.BlockSpec((1, 1, CH), lambda h, m: (h, 0, 0)),
        ],
        out_specs=pl.BlockSpec((bm, CH // 2), lambda h, m: (h * nm + m, 0)),
        out_shape=jax.ShapeDtypeStruct((2 * ne, CH // 2), jnp.int32),
        compiler_params=pltpu.CompilerParams(
            dimension_semantics=("parallel", "arbitrary")),
    )(edge_attr, WgT_s, bg_s)


def _make_sc_body(chunks):
    n_edges_h = chunks * E_BLK * NS
    ept = chunks * E_BLK

    def body(srcf_hbm, dstf_hbm, x2_hbm, gates_hbm, out_hbm,
             idx_v, rows0, rows1, g0, g1, acc,
             lsem0, lsem1, ssem0, ssem1, isem0, isem1, isem2, isem3):
        c = lax.axis_index("c")
        s = lax.axis_index("s")
        rows = (rows0, rows1)
        g = (g0, g1)
        lsem = (lsem0, lsem1)
        ssem = (ssem0, ssem1)
        isem = (isem0, isem1, isem2, isem3)

        def soff(k):
            return c * n_edges_h + s * ept + k * E_BLK

        def doff(k):
            return s * ept + k * E_BLK

        def idx_load(k, slot):
            pltpu.async_copy(srcf_hbm.at[pl.ds(soff(k), E_BLK)],
                             idx_v.at[slot], isem[slot])
            pltpu.async_copy(dstf_hbm.at[pl.ds(doff(k), E_BLK)],
                             idx_v.at[4 + slot], isem[slot])

        def idx_wait(k, slot):
            pltpu.make_async_copy(srcf_hbm.at[pl.ds(soff(k), E_BLK)],
                                  idx_v.at[slot], isem[slot]).wait()
            pltpu.make_async_copy(dstf_hbm.at[pl.ds(doff(k), E_BLK)],
                                  idx_v.at[4 + slot], isem[slot]).wait()

        def load_start(k, slot, b):
            pltpu.async_copy(x2_hbm.at[idx_v.at[slot]], rows[b], lsem[b])
            pltpu.async_copy(gates_hbm.at[pl.ds(soff(k), E_BLK)], g[b],
                             lsem[b])

        def load_wait(k, slot, b):
            pltpu.make_async_copy(x2_hbm.at[idx_v.at[slot]], rows[b],
                                  lsem[b]).wait()
            pltpu.make_async_copy(gates_hbm.at[pl.ds(soff(k), E_BLK)], g[b],
                                  lsem[b]).wait()

        def scat_start(slot, b):
            pltpu.async_copy(rows[b], acc.at[idx_v.at[4 + slot]], ssem[b],
                             add=True)

        def scat_wait(slot, b):
            pltpu.make_async_copy(rows[b], acc.at[idx_v.at[4 + slot]],
                                  ssem[b]).wait()

        def multiply(b):
            rv, gv = rows[b], g[b]

            @plsc.parallel_loop(0, E_BLK, unroll=2)
            def _(e):
                for t in range(CH // 32):
                    w = gv[e, pl.ds(t * 16, 16)]
                    ga = jax.lax.bitcast_convert_type(w << 16, jnp.float32)
                    gb = jax.lax.bitcast_convert_type(w & -65536, jnp.float32)
                    sl0 = pl.ds(t * 32, 16)
                    sl1 = pl.ds(t * 32 + 16, 16)
                    rv[e, sl0] = rv[e, sl0] * ga
                    rv[e, sl1] = rv[e, sl1] * gb

        @plsc.parallel_loop(0, E_BLK, unroll=2)
        def _(e):
            for j in range(CH // 16):
                rows0[e, pl.ds(j * 16, 16)] = jnp.zeros((16,), jnp.float32)

        def zacc(k, _):
            pltpu.sync_copy(rows0, acc.at[pl.ds(s * 640 + k * E_BLK, E_BLK)])
            return 0
        lax.fori_loop(0, ACC_ROWS // NS // E_BLK, zacc, 0)
        plsc.subcore_barrier()

        pltpu.sync_copy(srcf_hbm.at[pl.ds(soff(0), E_BLK)], idx_v.at[0])
        pltpu.sync_copy(dstf_hbm.at[pl.ds(doff(0), E_BLK)], idx_v.at[4])
        idx_load(1, 1)
        load_start(0, 0, 0)

        def substep(k, b4):
            b = b4 % 2
            load_wait(k, b4, b)

            @pl.when(k > 0)
            def _():
                scat_wait((b4 + 3) % 4, 1 - b)

            @pl.when(k < chunks - 2)
            def _():
                idx_load(k + 2, (b4 + 2) % 4)

            idx_wait(k + 1, (b4 + 1) % 4)
            load_start(k + 1, (b4 + 1) % 4, 1 - b)
            multiply(b)
            scat_start(b4, b)

        nq = (chunks - 1) // 4
        rem = (chunks - 1) - 4 * nq

        def quad(q, _):
            for b4 in range(4):
                substep(4 * q + b4, b4)
            return 0
        lax.fori_loop(0, nq, quad, 0)
        for b4 in range(rem):
            substep(4 * nq + b4, b4)

        kt = chunks - 1
        load_wait(kt, kt % 4, kt % 2)
        scat_wait((kt + 3) % 4, 1 - kt % 2)
        multiply(kt % 2)
        scat_start(kt % 4, kt % 2)
        scat_wait(kt % 4, kt % 2)
        plsc.subcore_barrier()

        @pl.when(s < NS - 1)
        def _():
            off = s * 624
            pltpu.sync_copy(acc.at[pl.ds(off, 624)],
                            out_hbm.at[pl.ds(c * N_NODES + off, 624)])

        @pl.when(s == NS - 1)
        def _():
            pltpu.sync_copy(acc.at[pl.ds(9360, 640)],
                            out_hbm.at[pl.ds(c * N_NODES + 9360, 640)])

    return body


def _sc_agg(srcf, dstf, x2, gates, chunks):
    mesh = plsc.VectorSubcoreMesh(core_axis_name="c", subcore_axis_name="s")
    fn = functools.partial(
        pl.kernel,
        mesh=mesh,
        out_type=jax.ShapeDtypeStruct((2 * N_NODES, CH), jnp.float32),
        scratch_types=[
            pltpu.VMEM((8, E_BLK), jnp.int32),
            pltpu.VMEM((E_BLK, CH), jnp.float32),
            pltpu.VMEM((E_BLK, CH), jnp.float32),
            pltpu.VMEM((E_BLK, CH // 2), jnp.int32),
            pltpu.VMEM((E_BLK, CH // 2), jnp.int32),
            pltpu.VMEM_SHARED((ACC_ROWS, CH), jnp.float32),
        ] + [pltpu.SemaphoreType.DMA] * 8,
    )(_make_sc_body(chunks))
    return fn(srcf, dstf, x2, gates)


def _combine_body(x_ref, a0a_ref, a1a_ref, a0b_ref, a1b_ref,
                  wc_ref, b_ref, out_ref):
    dn = (((1,), (1,)), ((), ()))
    w = wc_ref[...]
    acc = jax.lax.dot_general(x_ref[...], w[:, :256], dn,
                              preferred_element_type=jnp.float32)
    acc += jax.lax.dot_general(a0a_ref[...] + a0b_ref[...],
                               w[:, 256:256 + CH], dn,
                               preferred_element_type=jnp.float32)
    acc += jax.lax.dot_general(a1a_ref[...] + a1b_ref[...],
                               w[:, 256 + CH:], dn,
                               preferred_element_type=jnp.float32)
    out_ref[...] = acc + b_ref[...]


def _combine_tc(x, agg_a, agg_b, Wc, bc2):
    BM = 1000
    nb = N_NODES // BM
    return pl.pallas_call(
        _combine_body,
        grid=(nb,),
        in_specs=[
            pl.BlockSpec((BM, 256), lambda m: (m, 0)),
            pl.BlockSpec((BM, CH), lambda m: (m, 0)),
            pl.BlockSpec((BM, CH), lambda m: (m + nb, 0)),
            pl.BlockSpec((BM, CH), lambda m: (m, 0)),
            pl.BlockSpec((BM, CH), lambda m: (m + nb, 0)),
            pl.BlockSpec((256, 512), lambda m: (0, 0)),
            pl.BlockSpec((1, 256), lambda m: (0, 0)),
        ],
        out_specs=pl.BlockSpec((BM, 256), lambda m: (m, 0)),
        out_shape=jax.ShapeDtypeStruct((N_NODES, 256), jnp.float32),
    )(x, agg_a, agg_a, agg_b, agg_b, Wc, bc2)


def _edge_prep(src_h, dst_h):
    srcf = jnp.stack([2 * src_h, 2 * src_h + 1]).reshape(-1)
    return srcf, dst_h


def kernel(x, edge_index, edge_attr, Wg, bg, Wc, bc):
    src = edge_index[0].astype(jnp.int32)
    dst = edge_index[1].astype(jnp.int32)
    x2 = x.reshape(2 * N_NODES, CH)

    wi = jnp.arange(CH // 2)
    base = 32 * (wi // 16) + wi % 16
    perm = jnp.concatenate([base, base + 16])
    WgT_s = jnp.stack([Wg[:CH].T, Wg[CH:].T])[:, :, perm]
    bg_s = jnp.stack([bg[:CH], bg[CH:]])[:, perm].reshape(2, 1, CH)

    srcf_a, dstf_a = _edge_prep(src[:E_A], dst[:E_A])
    srcf_b, dstf_b = _edge_prep(src[E_A:], dst[E_A:])

    gates_a = _gates_tc(edge_attr[:E_A], WgT_s, bg_s, 5280)
    attr_b, = jax.lax.optimization_barrier((edge_attr[E_A:], gates_a))[:1]
    gates_b = _gates_tc(attr_b, WgT_s, bg_s, 4720)

    agg_a = _sc_agg(srcf_a, dstf_a, x2, gates_a, CHUNKS_A)
    agg_b = _sc_agg(srcf_b, dstf_b, x2, gates_b, CHUNKS_B)

    return _combine_tc(x, agg_a, agg_b, Wc, bc.reshape(1, 256))


# speedup vs baseline: 1.0899x; 1.0461x over previous
import functools

import jax
import jax.numpy as jnp
from jax import lax
from jax.experimental import pallas as pl
from jax.experimental.pallas import tpu as pltpu
from jax.experimental.pallas import tpu_sc as plsc

N_NODES = 10000
N_EDGES = 160000
CH = 128
E_BLK = 80
NS = 16
ACC_ROWS = 10240
CHUNKS_A = 45
CHUNKS_B = 80
E_A = CHUNKS_A * E_BLK * NS
E_B = CHUNKS_B * E_BLK * NS


def _gates_body(attr_ref, wT_ref, b_ref, out_ref):
    z = jnp.dot(attr_ref[...], wT_ref[0], preferred_element_type=jnp.float32)
    gp = jax.nn.sigmoid(z + b_ref[0])
    ib = jax.lax.bitcast_convert_type(gp, jnp.int32) + 0x8000
    lo = ib[:, :64] >> 16
    hi = ib[:, 64:] & (-65536)
    out_ref[...] = lo | hi


def _gates_tc(edge_attr, WgT_s, bg_s, bm):
    ne = edge_attr.shape[0]
    nm = ne // bm
    return pl.pallas_call(
        _gates_body,
        grid=(2, nm),
        in_specs=[
            pl.BlockSpec((bm, 16), lambda h, m: (m, 0)),
            pl.BlockSpec((1, 16, CH), lambda h, m: (h, 0, 0)),
            pl.BlockSpec((1, 1, CH), lambda h, m: (h, 0, 0)),
        ],
        out_specs=pl.BlockSpec((bm, CH // 2), lambda h, m: (h * nm + m, 0)),
        out_shape=jax.ShapeDtypeStruct((2 * ne, CH // 2), jnp.int32),
        compiler_params=pltpu.CompilerParams(
            dimension_semantics=("parallel", "arbitrary")),
    )(edge_attr, WgT_s, bg_s)


def _make_sc_body(chunks):
    n_edges_h = chunks * E_BLK * NS
    ept = chunks * E_BLK

    def body(srcf_hbm, dstf_hbm, x2_hbm, gates_hbm, out_hbm,
             idx_v, rows0, rows1, g0, g1, acc,
             lsem0, lsem1, ssem0, ssem1, isem0, isem1, isem2, isem3):
        c = lax.axis_index("c")
        s = lax.axis_index("s")
        rows = (rows0, rows1)
        g = (g0, g1)
        lsem = (lsem0, lsem1)
        ssem = (ssem0, ssem1)
        isem = (isem0, isem1, isem2, isem3)

        def soff(k):
            return c * n_edges_h + s * ept + k * E_BLK

        def doff(k):
            return s * ept + k * E_BLK

        def idx_load(k, slot):
            pltpu.async_copy(srcf_hbm.at[pl.ds(soff(k), E_BLK)],
                             idx_v.at[slot], isem[slot])
            pltpu.async_copy(dstf_hbm.at[pl.ds(doff(k), E_BLK)],
                             idx_v.at[4 + slot], isem[slot])

        def idx_wait(k, slot):
            pltpu.make_async_copy(srcf_hbm.at[pl.ds(soff(k), E_BLK)],
                                  idx_v.at[slot], isem[slot]).wait()
            pltpu.make_async_copy(dstf_hbm.at[pl.ds(doff(k), E_BLK)],
                                  idx_v.at[4 + slot], isem[slot]).wait()

        def load_start(k, slot, b):
            pltpu.async_copy(x2_hbm.at[idx_v.at[slot]], rows[b], lsem[b])
            pltpu.async_copy(gates_hbm.at[pl.ds(soff(k), E_BLK)], g[b],
                             lsem[b])

        def load_wait(k, slot, b):
            pltpu.make_async_copy(x2_hbm.at[idx_v.at[slot]], rows[b],
                                  lsem[b]).wait()
            pltpu.make_async_copy(gates_hbm.at[pl.ds(soff(k), E_BLK)], g[b],
                                  lsem[b]).wait()

        def scat_start(slot, b):
            pltpu.async_copy(rows[b], acc.at[idx_v.at[4 + slot]], ssem[b],
                             add=True)

        def scat_wait(slot, b):
            pltpu.make_async_copy(rows[b], acc.at[idx_v.at[4 + slot]],
                                  ssem[b]).wait()

        def multiply(b):
            rv, gv = rows[b], g[b]

            @plsc.parallel_loop(0, E_BLK, unroll=2)
            def _(e):
                for t in range(CH // 32):
                    w = gv[e, pl.ds(t * 16, 16)]
                    ga = jax.lax.bitcast_convert_type(w << 16, jnp.float32)
                    gb = jax.lax.bitcast_convert_type(w & -65536, jnp.float32)
                    sl0 = pl.ds(t * 32, 16)
                    sl1 = pl.ds(t * 32 + 16, 16)
                    rv[e, sl0] = rv[e, sl0] * ga
                    rv[e, sl1] = rv[e, sl1] * gb

        @plsc.parallel_loop(0, E_BLK, unroll=2)
        def _(e):
            for j in range(CH // 16):
                rows0[e, pl.ds(j * 16, 16)] = jnp.zeros((16,), jnp.float32)

        def zacc(k, _):
            pltpu.sync_copy(rows0, acc.at[pl.ds(s * 640 + k * E_BLK, E_BLK)])
            return 0
        lax.fori_loop(0, ACC_ROWS // NS // E_BLK, zacc, 0)
        plsc.subcore_barrier()

        pltpu.sync_copy(srcf_hbm.at[pl.ds(soff(0), E_BLK)], idx_v.at[0])
        pltpu.sync_copy(dstf_hbm.at[pl.ds(doff(0), E_BLK)], idx_v.at[4])
        idx_load(1, 1)
        load_start(0, 0, 0)

        def substep(k, b4):
            b = b4 % 2
            load_wait(k, b4, b)

            @pl.when(k > 0)
            def _():
                scat_wait((b4 + 3) % 4, 1 - b)

            @pl.when(k < chunks - 2)
            def _():
                idx_load(k + 2, (b4 + 2) % 4)

            idx_wait(k + 1, (b4 + 1) % 4)
            load_start(k + 1, (b4 + 1) % 4, 1 - b)
            multiply(b)
            scat_start(b4, b)

        nq = (chunks - 1) // 4
        rem = (chunks - 1) - 4 * nq

        def quad(q, _):
            for b4 in range(4):
                substep(4 * q + b4, b4)
            return 0
        lax.fori_loop(0, nq, quad, 0)
        for b4 in range(rem):
            substep(4 * nq + b4, b4)

        kt = chunks - 1
        load_wait(kt, kt % 4, kt % 2)
        scat_wait((kt + 3) % 4, 1 - kt % 2)
        multiply(kt % 2)
        scat_start(kt % 4, kt % 2)
        scat_wait(kt % 4, kt % 2)
        plsc.subcore_barrier()

        @pl.when(s < NS - 1)
        def _():
            off = s * 624
            pltpu.sync_copy(acc.at[pl.ds(off, 624)],
                            out_hbm.at[pl.ds(c * N_NODES + off, 624)])

        @pl.when(s == NS - 1)
        def _():
            pltpu.sync_copy(acc.at[pl.ds(9360, 640)],
                            out_hbm.at[pl.ds(c * N_NODES + 9360, 640)])

    return body


def _sc_agg(srcf, dstf, x2, gates, chunks):
    mesh = plsc.VectorSubcoreMesh(core_axis_name="c", subcore_axis_name="s")
    fn = functools.partial(
        pl.kernel,
        mesh=mesh,
        out_type=jax.ShapeDtypeStruct((2 * N_NODES, CH), jnp.float32),
        scratch_types=[
            pltpu.VMEM((8, E_BLK), jnp.int32),
            pltpu.VMEM((E_BLK, CH), jnp.float32),
            pltpu.VMEM((E_BLK, CH), jnp.float32),
            pltpu.VMEM((E_BLK, CH // 2), jnp.int32),
            pltpu.VMEM((E_BLK, CH // 2), jnp.int32),
            pltpu.VMEM_SHARED((ACC_ROWS, CH), jnp.float32),
        ] + [pltpu.SemaphoreType.DMA] * 8,
    )(_make_sc_body(chunks))
    return fn(srcf, dstf, x2, gates)


def _combine_body(x_ref, a0a_ref, a1a_ref, a0b_ref, a1b_ref,
                  wc_ref, b_ref, out_ref):
    dn = (((1,), (1,)), ((), ()))
    w = wc_ref[...]
    acc = jax.lax.dot_general(x_ref[...], w[:, :256], dn,
                              preferred_element_type=jnp.float32)
    acc += jax.lax.dot_general(a0a_ref[...] + a0b_ref[...],
                               w[:, 256:256 + CH], dn,
                               preferred_element_type=jnp.float32)
    acc += jax.lax.dot_general(a1a_ref[...] + a1b_ref[...],
                               w[:, 256 + CH:], dn,
                               preferred_element_type=jnp.float32)
    out_ref[...] = acc + b_ref[...]


def _combine_tc(x, agg_a, agg_b, Wc, bc2):
    BM = 1000
    nb = N_NODES // BM
    return pl.pallas_call(
        _combine_body,
        grid=(nb,),
        in_specs=[
            pl.BlockSpec((BM, 256), lambda m: (m, 0)),
            pl.BlockSpec((BM, CH), lambda m: (m, 0)),
            pl.BlockSpec((BM, CH), lambda m: (m + nb, 0)),
            pl.BlockSpec((BM, CH), lambda m: (m, 0)),
            pl.BlockSpec((BM, CH), lambda m: (m + nb, 0)),
            pl.BlockSpec((256, 512), lambda m: (0, 0)),
            pl.BlockSpec((1, 256), lambda m: (0, 0)),
        ],
        out_specs=pl.BlockSpec((BM, 256), lambda m: (m, 0)),
        out_shape=jax.ShapeDtypeStruct((N_NODES, 256), jnp.float32),
    )(x, agg_a, agg_a, agg_b, agg_b, Wc, bc2)


def _edge_prep(src_h, dst_h):
    srcf = jnp.stack([2 * src_h, 2 * src_h + 1]).reshape(-1)
    return srcf, dst_h


def kernel(x, edge_index, edge_attr, Wg, bg, Wc, bc):
    src = edge_index[0].astype(jnp.int32)
    dst = edge_index[1].astype(jnp.int32)
    x2 = x.reshape(2 * N_NODES, CH)

    wi = jnp.arange(CH // 2)
    base = 32 * (wi // 16) + wi % 16
    perm = jnp.concatenate([base, base + 16])
    WgT_s = jnp.stack([Wg[:CH].T, Wg[CH:].T])[:, :, perm]
    bg_s = jnp.stack([bg[:CH], bg[CH:]])[:, perm].reshape(2, 1, CH)

    srcf_a, dstf_a = _edge_prep(src[:E_A], dst[:E_A])
    srcf_b, dstf_b = _edge_prep(src[E_A:], dst[E_A:])

    gates_a = _gates_tc(edge_attr[:E_A], WgT_s, bg_s, 7200)
    attr_b, = jax.lax.optimization_barrier((edge_attr[E_A:], gates_a))[:1]
    gates_b = _gates_tc(attr_b, WgT_s, bg_s, 6400)

    agg_a = _sc_agg(srcf_a, dstf_a, x2, gates_a, CHUNKS_A)
    agg_b = _sc_agg(srcf_b, dstf_b, x2, gates_b, CHUNKS_B)

    return _combine_tc(x, agg_a, agg_b, Wc, bc.reshape(1, 256))


# speedup vs baseline: 1.0901x; 1.0002x over previous
import functools

import jax
import jax.numpy as jnp
from jax import lax
from jax.experimental import pallas as pl
from jax.experimental.pallas import tpu as pltpu
from jax.experimental.pallas import tpu_sc as plsc

N_NODES = 10000
N_EDGES = 160000
CH = 128
E_BLK = 80
NS = 16
ACC_ROWS = 10240
CHUNKS_A = 45
CHUNKS_B = 80
E_A = CHUNKS_A * E_BLK * NS
E_B = CHUNKS_B * E_BLK * NS


def _gates_body(attr_ref, wT_ref, b_ref, out_ref):
    z = jnp.dot(attr_ref[...], wT_ref[0], preferred_element_type=jnp.float32)
    gp = jax.nn.sigmoid(z + b_ref[0])
    ib = jax.lax.bitcast_convert_type(gp, jnp.int32) + 0x8000
    lo = ib[:, :64] >> 16
    hi = ib[:, 64:] & (-65536)
    out_ref[...] = lo | hi


def _gates_tc(edge_attr, WgT_s, bg_s, bm):
    ne = edge_attr.shape[0]
    nm = ne // bm
    return pl.pallas_call(
        _gates_body,
        grid=(2, nm),
        in_specs=[
            pl.BlockSpec((bm, 16), lambda h, m: (m, 0)),
            pl.BlockSpec((1, 16, CH), lambda h, m: (h, 0, 0)),
            pl.BlockSpec((1, 1, CH), lambda h, m: (h, 0, 0)),
        ],
        out_specs=pl.BlockSpec((bm, CH // 2), lambda h, m: (h * nm + m, 0)),
        out_shape=jax.ShapeDtypeStruct((2 * ne, CH // 2), jnp.int32),
        compiler_params=pltpu.CompilerParams(
            dimension_semantics=("parallel", "arbitrary")),
    )(edge_attr, WgT_s, bg_s)


def _make_sc_body(chunks):
    n_edges_h = chunks * E_BLK * NS
    ept = chunks * E_BLK

    def body(srcf_hbm, dstf_hbm, x2_hbm, gates_hbm, out_hbm,
             idx_v, rows0, rows1, g0, g1, acc,
             lsem0, lsem1, ssem0, ssem1, isem0, isem1, isem2, isem3):
        c = lax.axis_index("c")
        s = lax.axis_index("s")
        rows = (rows0, rows1)
        g = (g0, g1)
        lsem = (lsem0, lsem1)
        ssem = (ssem0, ssem1)
        isem = (isem0, isem1, isem2, isem3)

        def soff(k):
            return c * n_edges_h + s * ept + k * E_BLK

        def doff(k):
            return s * ept + k * E_BLK

        def idx_load(k, slot):
            pltpu.async_copy(srcf_hbm.at[pl.ds(soff(k), E_BLK)],
                             idx_v.at[slot], isem[slot])
            pltpu.async_copy(dstf_hbm.at[pl.ds(doff(k), E_BLK)],
                             idx_v.at[4 + slot], isem[slot])

        def idx_wait(k, slot):
            pltpu.make_async_copy(srcf_hbm.at[pl.ds(soff(k), E_BLK)],
                                  idx_v.at[slot], isem[slot]).wait()
            pltpu.make_async_copy(dstf_hbm.at[pl.ds(doff(k), E_BLK)],
                                  idx_v.at[4 + slot], isem[slot]).wait()

        def load_start(k, slot, b):
            pltpu.async_copy(x2_hbm.at[idx_v.at[slot]], rows[b], lsem[b])
            pltpu.async_copy(gates_hbm.at[pl.ds(soff(k), E_BLK)], g[b],
                             lsem[b])

        def load_wait(k, slot, b):
            pltpu.make_async_copy(x2_hbm.at[idx_v.at[slot]], rows[b],
                                  lsem[b]).wait()
            pltpu.make_async_copy(gates_hbm.at[pl.ds(soff(k), E_BLK)], g[b],
                                  lsem[b]).wait()

        def scat_start(slot, b):
            pltpu.async_copy(rows[b], acc.at[idx_v.at[4 + slot]], ssem[b],
                             add=True)

        def scat_wait(slot, b):
            pltpu.make_async_copy(rows[b], acc.at[idx_v.at[4 + slot]],
                                  ssem[b]).wait()

        def multiply(b):
            rv, gv = rows[b], g[b]

            @plsc.parallel_loop(0, E_BLK, unroll=4)
            def _(e):
                for t in range(CH // 32):
                    w = gv[e, pl.ds(t * 16, 16)]
                    ga = jax.lax.bitcast_convert_type(w << 16, jnp.float32)
                    gb = jax.lax.bitcast_convert_type(w & -65536, jnp.float32)
                    sl0 = pl.ds(t * 32, 16)
                    sl1 = pl.ds(t * 32 + 16, 16)
                    rv[e, sl0] = rv[e, sl0] * ga
                    rv[e, sl1] = rv[e, sl1] * gb

        @plsc.parallel_loop(0, E_BLK, unroll=2)
        def _(e):
            for j in range(CH // 16):
                rows0[e, pl.ds(j * 16, 16)] = jnp.zeros((16,), jnp.float32)

        def zacc(k, _):
            pltpu.sync_copy(rows0, acc.at[pl.ds(s * 640 + k * E_BLK, E_BLK)])
            return 0
        lax.fori_loop(0, ACC_ROWS // NS // E_BLK, zacc, 0)
        plsc.subcore_barrier()

        pltpu.sync_copy(srcf_hbm.at[pl.ds(soff(0), E_BLK)], idx_v.at[0])
        pltpu.sync_copy(dstf_hbm.at[pl.ds(doff(0), E_BLK)], idx_v.at[4])
        idx_load(1, 1)
        load_start(0, 0, 0)

        def substep(k, b4):
            b = b4 % 2
            load_wait(k, b4, b)

            @pl.when(k > 0)
            def _():
                scat_wait((b4 + 3) % 4, 1 - b)

            @pl.when(k < chunks - 2)
            def _():
                idx_load(k + 2, (b4 + 2) % 4)

            idx_wait(k + 1, (b4 + 1) % 4)
            load_start(k + 1, (b4 + 1) % 4, 1 - b)
            multiply(b)
            scat_start(b4, b)

        nq = (chunks - 1) // 4
        rem = (chunks - 1) - 4 * nq

        def quad(q, _):
            for b4 in range(4):
                substep(4 * q + b4, b4)
            return 0
        lax.fori_loop(0, nq, quad, 0)
        for b4 in range(rem):
            substep(4 * nq + b4, b4)

        kt = chunks - 1
        load_wait(kt, kt % 4, kt % 2)
        scat_wait((kt + 3) % 4, 1 - kt % 2)
        multiply(kt % 2)
        scat_start(kt % 4, kt % 2)
        scat_wait(kt % 4, kt % 2)
        plsc.subcore_barrier()

        @pl.when(s < NS - 1)
        def _():
            off = s * 624
            pltpu.sync_copy(acc.at[pl.ds(off, 624)],
                            out_hbm.at[pl.ds(c * N_NODES + off, 624)])

        @pl.when(s == NS - 1)
        def _():
            pltpu.sync_copy(acc.at[pl.ds(9360, 640)],
                            out_hbm.at[pl.ds(c * N_NODES + 9360, 640)])

    return body


def _sc_agg(srcf, dstf, x2, gates, chunks):
    mesh = plsc.VectorSubcoreMesh(core_axis_name="c", subcore_axis_name="s")
    fn = functools.partial(
        pl.kernel,
        mesh=mesh,
        out_type=jax.ShapeDtypeStruct((2 * N_NODES, CH), jnp.float32),
        scratch_types=[
            pltpu.VMEM((8, E_BLK), jnp.int32),
            pltpu.VMEM((E_BLK, CH), jnp.float32),
            pltpu.VMEM((E_BLK, CH), jnp.float32),
            pltpu.VMEM((E_BLK, CH // 2), jnp.int32),
            pltpu.VMEM((E_BLK, CH // 2), jnp.int32),
            pltpu.VMEM_SHARED((ACC_ROWS, CH), jnp.float32),
        ] + [pltpu.SemaphoreType.DMA] * 8,
    )(_make_sc_body(chunks))
    return fn(srcf, dstf, x2, gates)


def _combine_body(x_ref, a0a_ref, a1a_ref, a0b_ref, a1b_ref,
                  wc_ref, b_ref, out_ref):
    dn = (((1,), (1,)), ((), ()))
    w = wc_ref[...]
    acc = jax.lax.dot_general(x_ref[...], w[:, :256], dn,
                              preferred_element_type=jnp.float32)
    acc += jax.lax.dot_general(a0a_ref[...] + a0b_ref[...],
                               w[:, 256:256 + CH], dn,
                               preferred_element_type=jnp.float32)
    acc += jax.lax.dot_general(a1a_ref[...] + a1b_ref[...],
                               w[:, 256 + CH:], dn,
                               preferred_element_type=jnp.float32)
    out_ref[...] = acc + b_ref[...]


def _combine_tc(x, agg_a, agg_b, Wc, bc2):
    BM = 1000
    nb = N_NODES // BM
    return pl.pallas_call(
        _combine_body,
        grid=(nb,),
        in_specs=[
            pl.BlockSpec((BM, 256), lambda m: (m, 0)),
            pl.BlockSpec((BM, CH), lambda m: (m, 0)),
            pl.BlockSpec((BM, CH), lambda m: (m + nb, 0)),
            pl.BlockSpec((BM, CH), lambda m: (m, 0)),
            pl.BlockSpec((BM, CH), lambda m: (m + nb, 0)),
            pl.BlockSpec((256, 512), lambda m: (0, 0)),
            pl.BlockSpec((1, 256), lambda m: (0, 0)),
        ],
        out_specs=pl.BlockSpec((BM, 256), lambda m: (m, 0)),
        out_shape=jax.ShapeDtypeStruct((N_NODES, 256), jnp.float32),
    )(x, agg_a, agg_a, agg_b, agg_b, Wc, bc2)


def _edge_prep(src_h, dst_h):
    srcf = jnp.stack([2 * src_h, 2 * src_h + 1]).reshape(-1)
    return srcf, dst_h


def kernel(x, edge_index, edge_attr, Wg, bg, Wc, bc):
    src = edge_index[0].astype(jnp.int32)
    dst = edge_index[1].astype(jnp.int32)
    x2 = x.reshape(2 * N_NODES, CH)

    wi = jnp.arange(CH // 2)
    base = 32 * (wi // 16) + wi % 16
    perm = jnp.concatenate([base, base + 16])
    WgT_s = jnp.stack([Wg[:CH].T, Wg[CH:].T])[:, :, perm]
    bg_s = jnp.stack([bg[:CH], bg[CH:]])[:, perm].reshape(2, 1, CH)

    srcf_a, dstf_a = _edge_prep(src[:E_A], dst[:E_A])
    srcf_b, dstf_b = _edge_prep(src[E_A:], dst[E_A:])

    gates_a = _gates_tc(edge_attr[:E_A], WgT_s, bg_s, 7200)
    attr_b, = jax.lax.optimization_barrier((edge_attr[E_A:], gates_a))[:1]
    gates_b = _gates_tc(attr_b, WgT_s, bg_s, 6400)

    agg_a = _sc_agg(srcf_a, dstf_a, x2, gates_a, CHUNKS_A)
    agg_b = _sc_agg(srcf_b, dstf_b, x2, gates_b, CHUNKS_B)

    return _combine_tc(x, agg_a, agg_b, Wc, bc.reshape(1, 256))
